# Initial kernel scaffold; baseline (speedup 1.0000x reference)
import jax
import jax.numpy as jnp
from jax.experimental import pallas as pl


def kernel(x, edge_index, W1, b1, W2, b2):
    raise NotImplementedError("write your pallas kernel here")


# speedup vs baseline: 6.9121x; 6.9121x over previous
import functools

import jax
import jax.numpy as jnp
from jax import lax
from jax.experimental import pallas as pl
from jax.experimental.pallas import tpu as pltpu
from jax.experimental.pallas import tpu_sc as plsc

N = 10000
E = 320000
D_IN = 128
HID = 256
DC = 128
NC, NS = 2, 16
CHUNK = 128
NPAD = 10240
RPT = NPAD // NS


def _mesh():
    return plsc.VectorSubcoreMesh(core_axis_name="c", subcore_axis_name="s")


_EPS_D = E // NC
_EPT_D = _EPS_D // NS
_FULL_D = _EPT_D // CHUNK
_TAIL_D = _EPT_D - _FULL_D * CHUNK


@functools.partial(
    pl.kernel,
    out_type=jax.ShapeDtypeStruct((NC * NPAD, DC), jnp.float32),
    mesh=_mesh(),
    scratch_types=[
        pltpu.VMEM((CHUNK,), jnp.int32),
        pltpu.VMEM((_TAIL_D,), jnp.int32),
        pltpu.VMEM((CHUNK, DC), jnp.float32),
        pltpu.VMEM((_TAIL_D, DC), jnp.float32),
        pltpu.VMEM((CHUNK, DC), jnp.float32),
        pltpu.VMEM_SHARED((NPAD, DC), jnp.float32),
        pltpu.SemaphoreType.DMA,
    ],
)
def _deg_kernel(row_hbm, ones_hbm, zeros_hbm, out_hbm,
                idx, idx_t, ones_v, ones_t, stag, acc, sem):
    c = lax.axis_index("c")
    s = lax.axis_index("s")
    pltpu.sync_copy(ones_hbm, ones_v)
    pltpu.sync_copy(ones_hbm.at[pl.ds(0, _TAIL_D)], ones_t)
    pltpu.sync_copy(zeros_hbm, stag)
    r0 = s * RPT
    for t in range(5):
        pltpu.sync_copy(stag, acc.at[pl.ds(r0 + t * CHUNK, CHUNK)])
    plsc.subcore_barrier()

    ebase = c * _EPS_D + s * _EPT_D

    def body(j, _):
        off = ebase + j * CHUNK
        pltpu.sync_copy(row_hbm.at[pl.ds(off, CHUNK)], idx)
        pltpu.sync_copy(ones_v, acc.at[idx], add=True)
        return 0

    lax.fori_loop(0, _FULL_D, body, 0)
    if _TAIL_D:
        toff = ebase + _FULL_D * CHUNK
        pltpu.sync_copy(row_hbm.at[pl.ds(toff, _TAIL_D)], idx_t)
        pltpu.sync_copy(ones_t, acc.at[idx_t], add=True)
    plsc.subcore_barrier()

    ob = c * NPAD + r0
    for t in range(5):
        pltpu.sync_copy(acc.at[pl.ds(r0 + t * CHUNK, CHUNK)], stag)
        pltpu.sync_copy(stag, out_hbm.at[pl.ds(ob + t * CHUNK, CHUNK)])


def _make_prop(eps, rows_tab):
    ept = eps // NS
    full = ept // CHUNK
    tail = ept - full * CHUNK

    @functools.partial(
        pl.kernel,
        out_type=jax.ShapeDtypeStruct((NC * NPAD, DC), jnp.float32),
        mesh=_mesh(),
        scratch_types=[
            pltpu.VMEM((CHUNK,), jnp.int32),
            pltpu.VMEM((CHUNK,), jnp.int32),
            pltpu.VMEM((CHUNK, DC), jnp.float32),
            pltpu.VMEM((tail,), jnp.int32),
            pltpu.VMEM((tail,), jnp.int32),
            pltpu.VMEM((tail, DC), jnp.float32),
            pltpu.VMEM_SHARED((NPAD, DC), jnp.float32),
            pltpu.SemaphoreType.DMA,
        ],
    )
    def prop(table_hbm, rowg_hbm, colg_hbm, zeros_hbm, out_hbm,
             idx_r, idx_c, stag, idx_rt, idx_ct, stag_t, acc, sem):
        c = lax.axis_index("c")
        s = lax.axis_index("s")
        pltpu.sync_copy(zeros_hbm, stag)
        r0 = s * RPT
        for t in range(5):
            pltpu.sync_copy(stag, acc.at[pl.ds(r0 + t * CHUNK, CHUNK)])
        plsc.subcore_barrier()

        ebase = c * eps + s * ept

        def body(j, _):
            off = ebase + j * CHUNK
            pltpu.sync_copy(rowg_hbm.at[pl.ds(off, CHUNK)], idx_r)
            pltpu.sync_copy(colg_hbm.at[pl.ds(off, CHUNK)], idx_c)
            pltpu.async_copy(table_hbm.at[idx_r], stag, sem).wait()
            pltpu.sync_copy(stag, acc.at[idx_c], add=True)
            return 0

        lax.fori_loop(0, full, body, 0)
        if tail:
            toff = ebase + full * CHUNK
            pltpu.sync_copy(rowg_hbm.at[pl.ds(toff, tail)], idx_rt)
            pltpu.sync_copy(colg_hbm.at[pl.ds(toff, tail)], idx_ct)
            pltpu.async_copy(table_hbm.at[idx_rt], stag_t, sem).wait()
            pltpu.sync_copy(stag_t, acc.at[idx_ct], add=True)
        plsc.subcore_barrier()

        ob = c * NPAD + r0
        for t in range(5):
            pltpu.sync_copy(acc.at[pl.ds(r0 + t * CHUNK, CHUNK)], stag)
            pltpu.sync_copy(stag, out_hbm.at[pl.ds(ob + t * CHUNK, CHUNK)])

    return prop


_prop_es = _make_prop(E // 2, N)
_prop_cs = _make_prop(E, NC * N)


_B = 1000
_HP = jax.lax.Precision.HIGHEST


def _prep_body(degp_ref, x_ref, dis_ref, s0_ref):
    deg = degp_ref[0, :, 0:1] + degp_ref[1, :, 0:1]
    dis = jnp.where(deg > 0, lax.rsqrt(deg), 0.0)
    dis_ref[...] = dis
    s0_ref[...] = dis * x_ref[...]


def _prep(degp, x):
    return pl.pallas_call(
        _prep_body,
        grid=(N // _B,),
        in_specs=[
            pl.BlockSpec((2, _B, DC), lambda i: (0, i, 0)),
            pl.BlockSpec((_B, D_IN), lambda i: (i, 0)),
        ],
        out_specs=[
            pl.BlockSpec((_B, 1), lambda i: (i, 0)),
            pl.BlockSpec((_B, D_IN), lambda i: (i, 0)),
        ],
        out_shape=[
            jax.ShapeDtypeStruct((N, 1), jnp.float32),
            jax.ShapeDtypeStruct((N, D_IN), jnp.float32),
        ],
    )(degp, x)


def _mid1_body(u_ref, dis_ref, s_ref):
    dis = dis_ref[...]
    s_ref[...] = (-(dis * dis)) * (u_ref[0] + u_ref[1])


def _mid1(u, dis):
    return pl.pallas_call(
        _mid1_body,
        grid=(N // _B,),
        in_specs=[
            pl.BlockSpec((2, _B, DC), lambda i: (0, i, 0)),
            pl.BlockSpec((_B, 1), lambda i: (i, 0)),
        ],
        out_specs=pl.BlockSpec((_B, DC), lambda i: (i, 0)),
        out_shape=jax.ShapeDtypeStruct((N, DC), jnp.float32),
    )(u, dis)


def _mid2_body(u_ref, dis_ref, s_ref):
    dis = dis_ref[...]
    s_ref[0] = (-(dis * dis)) * u_ref[0]


def _mid2(u, dis):
    return pl.pallas_call(
        _mid2_body,
        grid=(2, N // _B),
        in_specs=[
            pl.BlockSpec((1, _B, DC), lambda c, i: (c, i, 0)),
            pl.BlockSpec((_B, 1), lambda c, i: (i, 0)),
        ],
        out_specs=pl.BlockSpec((1, _B, DC), lambda c, i: (c, i, 0)),
        out_shape=jax.ShapeDtypeStruct((2, N, DC), jnp.float32),
    )(u, dis)


def _layer1_body(x_ref, u1_ref, u2_ref, dis_ref, wm_ref, wa_ref, wb_ref,
                 b_ref, out_ref, sp_ref):
    dis = dis_ref[...]
    u1 = u1_ref[0] + u1_ref[1]
    u2 = u2_ref[0] + u2_ref[1]
    acc = jnp.dot(x_ref[...], wm_ref[...], precision=_HP,
                  preferred_element_type=jnp.float32)
    acc = acc + jnp.dot(-dis * u1, wa_ref[...], precision=_HP,
                        preferred_element_type=jnp.float32)
    acc = acc + jnp.dot(-2.0 * dis * u2, wb_ref[...], precision=_HP,
                        preferred_element_type=jnp.float32)
    h = jnp.maximum(acc + b_ref[...], 0.0)
    out_ref[...] = h
    sp_ref[0] = dis * h[:, :DC]
    sp_ref[1] = dis * h[:, DC:]


def _layer1(x, u1, u2, dis, wm, wa, wb, b):
    return pl.pallas_call(
        _layer1_body,
        grid=(N // _B,),
        in_specs=[
            pl.BlockSpec((_B, D_IN), lambda i: (i, 0)),
            pl.BlockSpec((2, _B, DC), lambda i: (0, i, 0)),
            pl.BlockSpec((2, _B, DC), lambda i: (0, i, 0)),
            pl.BlockSpec((_B, 1), lambda i: (i, 0)),
            pl.BlockSpec((D_IN, HID), lambda i: (0, 0)),
            pl.BlockSpec((D_IN, HID), lambda i: (0, 0)),
            pl.BlockSpec((D_IN, HID), lambda i: (0, 0)),
            pl.BlockSpec((1, HID), lambda i: (0, 0)),
        ],
        out_specs=[
            pl.BlockSpec((_B, HID), lambda i: (i, 0)),
            pl.BlockSpec((2, _B, DC), lambda i: (0, i, 0)),
        ],
        out_shape=[
            jax.ShapeDtypeStruct((N, HID), jnp.float32),
            jax.ShapeDtypeStruct((2, N, DC), jnp.float32),
        ],
    )(x, u1, u2, dis, wm, wa, wb, b)


def _layer2_body(h_ref, u1_ref, u2_ref, dis_ref, wm_ref, wa_ref, wb_ref,
                 b_ref, out_ref):
    dis = dis_ref[...]
    u1c = jnp.concatenate([u1_ref[0], u1_ref[1]], axis=1)
    u2c = jnp.concatenate([u2_ref[0], u2_ref[1]], axis=1)
    acc = jnp.dot(h_ref[...], wm_ref[...], precision=_HP,
                  preferred_element_type=jnp.float32)
    acc = acc + jnp.dot(-dis * u1c, wa_ref[...], precision=_HP,
                        preferred_element_type=jnp.float32)
    acc = acc + jnp.dot(-2.0 * dis * u2c, wb_ref[...], precision=_HP,
                        preferred_element_type=jnp.float32)
    out_ref[...] = jnp.maximum(acc + b_ref[...], 0.0)


def _layer2(h, u1, u2, dis, wm, wa, wb, b):
    return pl.pallas_call(
        _layer2_body,
        grid=(N // _B,),
        in_specs=[
            pl.BlockSpec((_B, HID), lambda i: (i, 0)),
            pl.BlockSpec((2, _B, DC), lambda i: (0, i, 0)),
            pl.BlockSpec((2, _B, DC), lambda i: (0, i, 0)),
            pl.BlockSpec((_B, 1), lambda i: (i, 0)),
            pl.BlockSpec((HID, HID), lambda i: (0, 0)),
            pl.BlockSpec((HID, HID), lambda i: (0, 0)),
            pl.BlockSpec((HID, HID), lambda i: (0, 0)),
            pl.BlockSpec((1, HID), lambda i: (0, 0)),
        ],
        out_specs=pl.BlockSpec((_B, HID), lambda i: (i, 0)),
        out_shape=jax.ShapeDtypeStruct((N, HID), jnp.float32),
    )(h, u1, u2, dis, wm, wa, wb, b)


def kernel(x, edge_index, W1, b1, W2, b2):
    row = edge_index[0].astype(jnp.int32)
    col = edge_index[1].astype(jnp.int32)
    rowg1 = row
    colg1 = col
    rowg2 = jnp.concatenate([row, row + N])
    colg2 = jnp.concatenate([col, col])

    ones128 = jnp.ones((CHUNK, DC), jnp.float32)
    zeros128 = jnp.zeros((CHUNK, DC), jnp.float32)

    degp = _deg_kernel(row, ones128, zeros128)
    degp = degp.reshape(NC, NPAD, DC)

    dis, s0 = _prep(degp, x)

    u1 = _prop_es(s0, rowg1, colg1, zeros128)
    u1 = u1.reshape(2, NPAD, DC)[:, :N, :]
    s1 = _mid1(u1, dis)
    u2 = _prop_es(s1, rowg1, colg1, zeros128)
    u2 = u2.reshape(2, NPAD, DC)[:, :N, :]

    w1m = W1[0] - W1[2]
    h, s0p = _layer1(x, u1, u2, dis, w1m, W1[1], W1[2], b1.reshape(1, HID))

    u1p = _prop_cs(s0p.reshape(NC * N, DC), rowg2, colg2, zeros128)
    u1p = u1p.reshape(2, NPAD, DC)[:, :N, :]
    s1p = _mid2(u1p, dis)
    u2p = _prop_cs(s1p.reshape(NC * N, DC), rowg2, colg2, zeros128)
    u2p = u2p.reshape(2, NPAD, DC)[:, :N, :]

    w2m = W2[0] - W2[2]
    out = _layer2(h, u1p, u2p, dis, w2m, W2[1], W2[2], b2.reshape(1, HID))
    return out


# speedup vs baseline: 12.2091x; 1.7663x over previous
import functools

import jax
import jax.numpy as jnp
from jax import lax
from jax.experimental import pallas as pl
from jax.experimental.pallas import tpu as pltpu
from jax.experimental.pallas import tpu_sc as plsc

N = 10000
E = 320000
D_IN = 128
HID = 256
DC = 128
NC, NS = 2, 16
CHUNK = 128
NPAD = 10240
RPT = NPAD // NS


def _mesh():
    return plsc.VectorSubcoreMesh(core_axis_name="c", subcore_axis_name="s")


_EPS_D = E // NC
_EPT_D = _EPS_D // NS
_FULL_D = _EPT_D // CHUNK
_TAIL_D = _EPT_D - _FULL_D * CHUNK


@functools.partial(
    pl.kernel,
    out_type=jax.ShapeDtypeStruct((NC * NPAD, DC), jnp.float32),
    mesh=_mesh(),
    scratch_types=[
        pltpu.VMEM((CHUNK,), jnp.int32),
        pltpu.VMEM((_TAIL_D,), jnp.int32),
        pltpu.VMEM((CHUNK, DC), jnp.float32),
        pltpu.VMEM((_TAIL_D, DC), jnp.float32),
        pltpu.VMEM((CHUNK, DC), jnp.float32),
        pltpu.VMEM_SHARED((NPAD, DC), jnp.float32),
        pltpu.SemaphoreType.DMA,
    ],
)
def _deg_kernel(row_hbm, ones_hbm, zeros_hbm, out_hbm,
                idx, idx_t, ones_v, ones_t, stag, acc, sem):
    c = lax.axis_index("c")
    s = lax.axis_index("s")
    pltpu.sync_copy(ones_hbm, ones_v)
    pltpu.sync_copy(ones_hbm.at[pl.ds(0, _TAIL_D)], ones_t)
    pltpu.sync_copy(zeros_hbm, stag)
    r0 = s * RPT
    for t in range(5):
        pltpu.sync_copy(stag, acc.at[pl.ds(r0 + t * CHUNK, CHUNK)])
    plsc.subcore_barrier()

    ebase = c * _EPS_D + s * _EPT_D

    def body(j, _):
        off = ebase + j * CHUNK
        pltpu.sync_copy(row_hbm.at[pl.ds(off, CHUNK)], idx)
        pltpu.sync_copy(ones_v, acc.at[idx], add=True)
        return 0

    lax.fori_loop(0, _FULL_D, body, 0)
    if _TAIL_D:
        toff = ebase + _FULL_D * CHUNK
        pltpu.sync_copy(row_hbm.at[pl.ds(toff, _TAIL_D)], idx_t)
        pltpu.sync_copy(ones_t, acc.at[idx_t], add=True)
    plsc.subcore_barrier()

    ob = c * NPAD + r0
    for t in range(5):
        pltpu.sync_copy(acc.at[pl.ds(r0 + t * CHUNK, CHUNK)], stag)
        pltpu.sync_copy(stag, out_hbm.at[pl.ds(ob + t * CHUNK, CHUNK)])


NB = 2
NG = 16


def _make_prop(cpt):
    @functools.partial(
        pl.kernel,
        out_type=jax.ShapeDtypeStruct((NC * NPAD, DC), jnp.float32),
        mesh=_mesh(),
        scratch_types=[
            pltpu.VMEM((NG * CHUNK,), jnp.int32),
            pltpu.VMEM((NG, CHUNK), jnp.int32),
            pltpu.VMEM((CHUNK, DC), jnp.float32),
            pltpu.VMEM((CHUNK, DC), jnp.float32),
            pltpu.VMEM_SHARED((NPAD, DC), jnp.float32),
            pltpu.SemaphoreType.DMA,
            pltpu.SemaphoreType.DMA,
            pltpu.SemaphoreType.DMA,
            pltpu.SemaphoreType.DMA,
        ],
    )
    def prop(table_hbm, rowg_hbm, colg_hbm, zeros_hbm, out_hbm,
             idxr_v, idxc_v, st0, st1, acc, g0, g1, s0, s1):
        stags = (st0, st1)
        gsems = (g0, g1)
        ssems = (s0, s1)
        c = lax.axis_index("c")
        s = lax.axis_index("s")
        pltpu.sync_copy(zeros_hbm, st0)
        r0 = s * RPT
        for t in range(5):
            pltpu.sync_copy(st0, acc.at[pl.ds(r0 + t * CHUNK, CHUNK)])
        tb = (c * NS + s) * cpt
        plsc.subcore_barrier()

        def gather(j, b):
            pltpu.async_copy(
                table_hbm.at[idxr_v.at[pl.ds(j * CHUNK, CHUNK)]],
                stags[b], gsems[b])

        def scatter(j, b):
            pltpu.async_copy(stags[b], acc.at[idxc_v.at[j]], ssems[b],
                             add=True)

        def drain(sem, b):
            pltpu.make_async_copy(zeros_hbm, stags[b], sem).wait()

        def group(g, _):
            gb = tb + g * NG
            pltpu.sync_copy(rowg_hbm.at[pl.ds(gb * CHUNK, NG * CHUNK)],
                            idxr_v)
            pltpu.sync_copy(colg_hbm.at[pl.ds(gb, NG)], idxc_v)
            for b in range(NB):
                gather(b, b)
            for j in range(NG):
                b = j % NB
                drain(gsems[b], b)
                scatter(j, b)
                if j + NB < NG:
                    drain(ssems[b], b)
                    gather(j + NB, b)
            for b in range(NB):
                drain(ssems[b], b)
            return 0

        lax.fori_loop(0, cpt // NG, group, 0)
        plsc.subcore_barrier()

        ob = c * NPAD + r0
        for t in range(5):
            pltpu.sync_copy(acc.at[pl.ds(r0 + t * CHUNK, CHUNK)], st0)
            pltpu.sync_copy(st0, out_hbm.at[pl.ds(ob + t * CHUNK, CHUNK)])

    return prop


CPT_ES = 80
CPT_CS = 160
LES = CPT_ES * CHUNK * NS
LCS = CPT_CS * CHUNK * NS
_prop_es = _make_prop(CPT_ES)
_prop_cs = _make_prop(CPT_CS)


_B = 1000
_HP = jax.lax.Precision.HIGHEST


def _prep_body(degp_ref, x_ref, dis_ref, s0_ref):
    deg = degp_ref[0, :, 0:1] + degp_ref[1, :, 0:1]
    dis = jnp.where(deg > 0, lax.rsqrt(deg), 0.0)
    dis_ref[...] = dis
    s0_ref[...] = dis * x_ref[...]


def _prep(degp, x):
    return pl.pallas_call(
        _prep_body,
        grid=(N // _B,),
        in_specs=[
            pl.BlockSpec((2, _B, DC), lambda i: (0, i, 0)),
            pl.BlockSpec((_B, D_IN), lambda i: (i, 0)),
        ],
        out_specs=[
            pl.BlockSpec((_B, 1), lambda i: (i, 0)),
            pl.BlockSpec((_B, D_IN), lambda i: (i, 0)),
        ],
        out_shape=[
            jax.ShapeDtypeStruct((N, 1), jnp.float32),
            jax.ShapeDtypeStruct((N, D_IN), jnp.float32),
        ],
    )(degp, x)


def _mid1_body(u_ref, dis_ref, s_ref):
    dis = dis_ref[...]
    s_ref[...] = (-(dis * dis)) * (u_ref[0] + u_ref[1])


def _mid1(u, dis):
    return pl.pallas_call(
        _mid1_body,
        grid=(N // _B,),
        in_specs=[
            pl.BlockSpec((2, _B, DC), lambda i: (0, i, 0)),
            pl.BlockSpec((_B, 1), lambda i: (i, 0)),
        ],
        out_specs=pl.BlockSpec((_B, DC), lambda i: (i, 0)),
        out_shape=jax.ShapeDtypeStruct((N, DC), jnp.float32),
    )(u, dis)


def _mid2_body(u_ref, dis_ref, s_ref):
    dis = dis_ref[...]
    s_ref[0] = (-(dis * dis)) * u_ref[0]


def _mid2(u, dis):
    return pl.pallas_call(
        _mid2_body,
        grid=(2, N // _B),
        in_specs=[
            pl.BlockSpec((1, _B, DC), lambda c, i: (c, i, 0)),
            pl.BlockSpec((_B, 1), lambda c, i: (i, 0)),
        ],
        out_specs=pl.BlockSpec((1, _B, DC), lambda c, i: (c, i, 0)),
        out_shape=jax.ShapeDtypeStruct((2, N, DC), jnp.float32),
    )(u, dis)


def _layer1_body(x_ref, u1_ref, u2_ref, dis_ref, wm_ref, wa_ref, wb_ref,
                 b_ref, out_ref, sp_ref):
    dis = dis_ref[...]
    u1 = u1_ref[0] + u1_ref[1]
    u2 = u2_ref[0] + u2_ref[1]
    acc = jnp.dot(x_ref[...], wm_ref[...], precision=_HP,
                  preferred_element_type=jnp.float32)
    acc = acc + jnp.dot(-dis * u1, wa_ref[...], precision=_HP,
                        preferred_element_type=jnp.float32)
    acc = acc + jnp.dot(-2.0 * dis * u2, wb_ref[...], precision=_HP,
                        preferred_element_type=jnp.float32)
    h = jnp.maximum(acc + b_ref[...], 0.0)
    out_ref[...] = h
    sp_ref[0] = dis * h[:, :DC]
    sp_ref[1] = dis * h[:, DC:]


def _layer1(x, u1, u2, dis, wm, wa, wb, b):
    return pl.pallas_call(
        _layer1_body,
        grid=(N // _B,),
        in_specs=[
            pl.BlockSpec((_B, D_IN), lambda i: (i, 0)),
            pl.BlockSpec((2, _B, DC), lambda i: (0, i, 0)),
            pl.BlockSpec((2, _B, DC), lambda i: (0, i, 0)),
            pl.BlockSpec((_B, 1), lambda i: (i, 0)),
            pl.BlockSpec((D_IN, HID), lambda i: (0, 0)),
            pl.BlockSpec((D_IN, HID), lambda i: (0, 0)),
            pl.BlockSpec((D_IN, HID), lambda i: (0, 0)),
            pl.BlockSpec((1, HID), lambda i: (0, 0)),
        ],
        out_specs=[
            pl.BlockSpec((_B, HID), lambda i: (i, 0)),
            pl.BlockSpec((2, _B, DC), lambda i: (0, i, 0)),
        ],
        out_shape=[
            jax.ShapeDtypeStruct((N, HID), jnp.float32),
            jax.ShapeDtypeStruct((2, N, DC), jnp.float32),
        ],
    )(x, u1, u2, dis, wm, wa, wb, b)


def _layer2_body(h_ref, u1_ref, u2_ref, dis_ref, wm_ref, wa_ref, wb_ref,
                 b_ref, out_ref):
    dis = dis_ref[...]
    u1c = jnp.concatenate([u1_ref[0], u1_ref[1]], axis=1)
    u2c = jnp.concatenate([u2_ref[0], u2_ref[1]], axis=1)
    acc = jnp.dot(h_ref[...], wm_ref[...], precision=_HP,
                  preferred_element_type=jnp.float32)
    acc = acc + jnp.dot(-dis * u1c, wa_ref[...], precision=_HP,
                        preferred_element_type=jnp.float32)
    acc = acc + jnp.dot(-2.0 * dis * u2c, wb_ref[...], precision=_HP,
                        preferred_element_type=jnp.float32)
    out_ref[...] = jnp.maximum(acc + b_ref[...], 0.0)


def _layer2(h, u1, u2, dis, wm, wa, wb, b):
    return pl.pallas_call(
        _layer2_body,
        grid=(N // _B,),
        in_specs=[
            pl.BlockSpec((_B, HID), lambda i: (i, 0)),
            pl.BlockSpec((2, _B, DC), lambda i: (0, i, 0)),
            pl.BlockSpec((2, _B, DC), lambda i: (0, i, 0)),
            pl.BlockSpec((_B, 1), lambda i: (i, 0)),
            pl.BlockSpec((HID, HID), lambda i: (0, 0)),
            pl.BlockSpec((HID, HID), lambda i: (0, 0)),
            pl.BlockSpec((HID, HID), lambda i: (0, 0)),
            pl.BlockSpec((1, HID), lambda i: (0, 0)),
        ],
        out_specs=pl.BlockSpec((_B, HID), lambda i: (i, 0)),
        out_shape=jax.ShapeDtypeStruct((N, HID), jnp.float32),
    )(h, u1, u2, dis, wm, wa, wb, b)


def kernel(x, edge_index, W1, b1, W2, b2):
    row = edge_index[0].astype(jnp.int32)
    col = edge_index[1].astype(jnp.int32)
    e2 = E // 2
    pes = LES - e2
    padr1 = jnp.arange(pes, dtype=jnp.int32) % N
    padc1 = N + jnp.arange(pes, dtype=jnp.int32) % 128
    rowg1 = jnp.concatenate([row[:e2], padr1, row[e2:], padr1])
    colg1 = jnp.concatenate([col[:e2], padc1, col[e2:], padc1])
    colg1 = colg1.reshape(-1, CHUNK)
    pcs = LCS - E
    padr2 = jnp.arange(pcs, dtype=jnp.int32) % N
    padc2 = N + jnp.arange(pcs, dtype=jnp.int32) % 128
    rowg2 = jnp.concatenate([row, padr2, row + N, padr2 + N])
    colg2 = jnp.concatenate([col, padc2, col, padc2]).reshape(-1, CHUNK)

    ones128 = jnp.ones((CHUNK, DC), jnp.float32)
    zeros128 = jnp.zeros((CHUNK, DC), jnp.float32)

    degp = _deg_kernel(row, ones128, zeros128)
    degp = degp.reshape(NC, NPAD, DC)

    dis, s0 = _prep(degp, x)

    u1 = _prop_es(s0, rowg1, colg1, zeros128)
    u1 = u1.reshape(2, NPAD, DC)[:, :N, :]
    s1 = _mid1(u1, dis)
    u2 = _prop_es(s1, rowg1, colg1, zeros128)
    u2 = u2.reshape(2, NPAD, DC)[:, :N, :]

    w1m = W1[0] - W1[2]
    h, s0p = _layer1(x, u1, u2, dis, w1m, W1[1], W1[2], b1.reshape(1, HID))

    u1p = _prop_cs(s0p.reshape(NC * N, DC), rowg2, colg2, zeros128)
    u1p = u1p.reshape(2, NPAD, DC)[:, :N, :]
    s1p = _mid2(u1p, dis)
    u2p = _prop_cs(s1p.reshape(NC * N, DC), rowg2, colg2, zeros128)
    u2p = u2p.reshape(2, NPAD, DC)[:, :N, :]

    w2m = W2[0] - W2[2]
    out = _layer2(h, u1p, u2p, dis, w2m, W2[1], W2[2], b2.reshape(1, HID))
    return out


# speedup vs baseline: 12.5464x; 1.0276x over previous
import functools

import jax
import jax.numpy as jnp
from jax import lax
from jax.experimental import pallas as pl
from jax.experimental.pallas import tpu as pltpu
from jax.experimental.pallas import tpu_sc as plsc

N = 10000
E = 320000
D_IN = 128
HID = 256
DC = 128
NC, NS = 2, 16
CHUNK = 128
NPAD = 10240
RPT = NPAD // NS


def _mesh():
    return plsc.VectorSubcoreMesh(core_axis_name="c", subcore_axis_name="s")


_CPT_D = 80
_NG_D = 16


@functools.partial(
    pl.kernel,
    out_type=jax.ShapeDtypeStruct((NC * NPAD, DC), jnp.float32),
    mesh=_mesh(),
    scratch_types=[
        pltpu.VMEM((_NG_D, CHUNK), jnp.int32),
        pltpu.VMEM((CHUNK, DC), jnp.float32),
        pltpu.VMEM((CHUNK, DC), jnp.float32),
        pltpu.VMEM_SHARED((NPAD, DC), jnp.float32),
        pltpu.SemaphoreType.DMA,
    ],
)
def _deg_kernel(rowsc_hbm, ones_hbm, zeros_hbm, out_hbm,
                idxc_v, ones_v, stag, acc, sem):
    c = lax.axis_index("c")
    s = lax.axis_index("s")
    pltpu.sync_copy(ones_hbm, ones_v)
    pltpu.sync_copy(zeros_hbm, stag)
    r0 = s * RPT
    for t in range(5):
        pltpu.sync_copy(stag, acc.at[pl.ds(r0 + t * CHUNK, CHUNK)])
    tb = (c * NS + s) * _CPT_D
    plsc.subcore_barrier()

    def group(g, _):
        pltpu.sync_copy(rowsc_hbm.at[pl.ds(tb + g * _NG_D, _NG_D)], idxc_v)
        for j in range(_NG_D):
            pltpu.async_copy(ones_v, acc.at[idxc_v.at[j]], sem, add=True)
        for j in range(_NG_D):
            pltpu.make_async_copy(zeros_hbm, stag, sem).wait()
        return 0

    lax.fori_loop(0, _CPT_D // _NG_D, group, 0)
    plsc.subcore_barrier()

    ob = c * NPAD + r0
    for t in range(5):
        pltpu.sync_copy(acc.at[pl.ds(r0 + t * CHUNK, CHUNK)], stag)
        pltpu.sync_copy(stag, out_hbm.at[pl.ds(ob + t * CHUNK, CHUNK)])


NB = 2
NG = 16


def _make_prop(cpt):
    @functools.partial(
        pl.kernel,
        out_type=jax.ShapeDtypeStruct((NC * NPAD, DC), jnp.float32),
        mesh=_mesh(),
        scratch_types=[
            pltpu.VMEM((NG * CHUNK,), jnp.int32),
            pltpu.VMEM((NG, CHUNK), jnp.int32),
            pltpu.VMEM((CHUNK, DC), jnp.float32),
            pltpu.VMEM((CHUNK, DC), jnp.float32),
            pltpu.VMEM_SHARED((NPAD, DC), jnp.float32),
            pltpu.SemaphoreType.DMA,
            pltpu.SemaphoreType.DMA,
            pltpu.SemaphoreType.DMA,
            pltpu.SemaphoreType.DMA,
        ],
    )
    def prop(table_hbm, rowg_hbm, colg_hbm, zeros_hbm, out_hbm,
             idxr_v, idxc_v, st0, st1, acc, g0, g1, s0, s1):
        stags = (st0, st1)
        gsems = (g0, g1)
        ssems = (s0, s1)
        c = lax.axis_index("c")
        s = lax.axis_index("s")
        pltpu.sync_copy(zeros_hbm, st0)
        r0 = s * RPT
        for t in range(5):
            pltpu.sync_copy(st0, acc.at[pl.ds(r0 + t * CHUNK, CHUNK)])
        tb = (c * NS + s) * cpt
        plsc.subcore_barrier()

        def gather(j, b):
            pltpu.async_copy(
                table_hbm.at[idxr_v.at[pl.ds(j * CHUNK, CHUNK)]],
                stags[b], gsems[b])

        def scatter(j, b):
            pltpu.async_copy(stags[b], acc.at[idxc_v.at[j]], ssems[b],
                             add=True)

        def drain(sem, b):
            pltpu.make_async_copy(zeros_hbm, stags[b], sem).wait()

        def group(g, _):
            gb = tb + g * NG
            pltpu.sync_copy(rowg_hbm.at[pl.ds(gb * CHUNK, NG * CHUNK)],
                            idxr_v)
            pltpu.sync_copy(colg_hbm.at[pl.ds(gb, NG)], idxc_v)
            for b in range(NB):
                gather(b, b)
            for j in range(NG):
                b = j % NB
                drain(gsems[b], b)
                scatter(j, b)
                if j + NB < NG:
                    drain(ssems[b], b)
                    gather(j + NB, b)
            for b in range(NB):
                drain(ssems[b], b)
            return 0

        lax.fori_loop(0, cpt // NG, group, 0)
        plsc.subcore_barrier()

        ob = c * NPAD + r0
        for t in range(5):
            pltpu.sync_copy(acc.at[pl.ds(r0 + t * CHUNK, CHUNK)], st0)
            pltpu.sync_copy(st0, out_hbm.at[pl.ds(ob + t * CHUNK, CHUNK)])

    return prop


CPT_ES = 80
CPT_CS = 160
LES = CPT_ES * CHUNK * NS
LCS = CPT_CS * CHUNK * NS
_prop_es = _make_prop(CPT_ES)
_prop_cs = _make_prop(CPT_CS)


_B = 1000
_HP = jax.lax.Precision.HIGHEST


def _prep_body(degp_ref, x_ref, dis_ref, s0_ref):
    deg = degp_ref[0, :, 0:1] + degp_ref[1, :, 0:1]
    dis = jnp.where(deg > 0, lax.rsqrt(deg), 0.0)
    dis_ref[...] = dis
    s0_ref[...] = dis * x_ref[...]


def _prep(degp, x):
    return pl.pallas_call(
        _prep_body,
        grid=(N // _B,),
        in_specs=[
            pl.BlockSpec((2, _B, DC), lambda i: (0, i, 0)),
            pl.BlockSpec((_B, D_IN), lambda i: (i, 0)),
        ],
        out_specs=[
            pl.BlockSpec((_B, 1), lambda i: (i, 0)),
            pl.BlockSpec((_B, D_IN), lambda i: (i, 0)),
        ],
        out_shape=[
            jax.ShapeDtypeStruct((N, 1), jnp.float32),
            jax.ShapeDtypeStruct((N, D_IN), jnp.float32),
        ],
    )(degp, x)


def _mid1_body(u_ref, dis_ref, s_ref):
    dis = dis_ref[...]
    s_ref[...] = (-(dis * dis)) * (u_ref[0] + u_ref[1])


def _mid1(u, dis):
    return pl.pallas_call(
        _mid1_body,
        grid=(N // _B,),
        in_specs=[
            pl.BlockSpec((2, _B, DC), lambda i: (0, i, 0)),
            pl.BlockSpec((_B, 1), lambda i: (i, 0)),
        ],
        out_specs=pl.BlockSpec((_B, DC), lambda i: (i, 0)),
        out_shape=jax.ShapeDtypeStruct((N, DC), jnp.float32),
    )(u, dis)


def _mid2_body(u_ref, dis_ref, s_ref):
    dis = dis_ref[...]
    s_ref[0] = (-(dis * dis)) * u_ref[0]


def _mid2(u, dis):
    return pl.pallas_call(
        _mid2_body,
        grid=(2, N // _B),
        in_specs=[
            pl.BlockSpec((1, _B, DC), lambda c, i: (c, i, 0)),
            pl.BlockSpec((_B, 1), lambda c, i: (i, 0)),
        ],
        out_specs=pl.BlockSpec((1, _B, DC), lambda c, i: (c, i, 0)),
        out_shape=jax.ShapeDtypeStruct((2, N, DC), jnp.float32),
    )(u, dis)


def _layer1_body(x_ref, u1_ref, u2_ref, dis_ref, wm_ref, wa_ref, wb_ref,
                 b_ref, out_ref, sp_ref):
    dis = dis_ref[...]
    u1 = u1_ref[0] + u1_ref[1]
    u2 = u2_ref[0] + u2_ref[1]
    acc = jnp.dot(x_ref[...], wm_ref[...], precision=_HP,
                  preferred_element_type=jnp.float32)
    acc = acc + jnp.dot(-dis * u1, wa_ref[...], precision=_HP,
                        preferred_element_type=jnp.float32)
    acc = acc + jnp.dot(-2.0 * dis * u2, wb_ref[...], precision=_HP,
                        preferred_element_type=jnp.float32)
    h = jnp.maximum(acc + b_ref[...], 0.0)
    out_ref[...] = h
    sp_ref[0] = dis * h[:, :DC]
    sp_ref[1] = dis * h[:, DC:]


def _layer1(x, u1, u2, dis, wm, wa, wb, b):
    return pl.pallas_call(
        _layer1_body,
        grid=(N // _B,),
        in_specs=[
            pl.BlockSpec((_B, D_IN), lambda i: (i, 0)),
            pl.BlockSpec((2, _B, DC), lambda i: (0, i, 0)),
            pl.BlockSpec((2, _B, DC), lambda i: (0, i, 0)),
            pl.BlockSpec((_B, 1), lambda i: (i, 0)),
            pl.BlockSpec((D_IN, HID), lambda i: (0, 0)),
            pl.BlockSpec((D_IN, HID), lambda i: (0, 0)),
            pl.BlockSpec((D_IN, HID), lambda i: (0, 0)),
            pl.BlockSpec((1, HID), lambda i: (0, 0)),
        ],
        out_specs=[
            pl.BlockSpec((_B, HID), lambda i: (i, 0)),
            pl.BlockSpec((2, _B, DC), lambda i: (0, i, 0)),
        ],
        out_shape=[
            jax.ShapeDtypeStruct((N, HID), jnp.float32),
            jax.ShapeDtypeStruct((2, N, DC), jnp.float32),
        ],
    )(x, u1, u2, dis, wm, wa, wb, b)


def _layer2_body(h_ref, u1_ref, u2_ref, dis_ref, wm_ref, wa_ref, wb_ref,
                 b_ref, out_ref):
    dis = dis_ref[...]
    u1c = jnp.concatenate([u1_ref[0], u1_ref[1]], axis=1)
    u2c = jnp.concatenate([u2_ref[0], u2_ref[1]], axis=1)
    acc = jnp.dot(h_ref[...], wm_ref[...], precision=_HP,
                  preferred_element_type=jnp.float32)
    acc = acc + jnp.dot(-dis * u1c, wa_ref[...], precision=_HP,
                        preferred_element_type=jnp.float32)
    acc = acc + jnp.dot(-2.0 * dis * u2c, wb_ref[...], precision=_HP,
                        preferred_element_type=jnp.float32)
    out_ref[...] = jnp.maximum(acc + b_ref[...], 0.0)


def _layer2(h, u1, u2, dis, wm, wa, wb, b):
    return pl.pallas_call(
        _layer2_body,
        grid=(N // _B,),
        in_specs=[
            pl.BlockSpec((_B, HID), lambda i: (i, 0)),
            pl.BlockSpec((2, _B, DC), lambda i: (0, i, 0)),
            pl.BlockSpec((2, _B, DC), lambda i: (0, i, 0)),
            pl.BlockSpec((_B, 1), lambda i: (i, 0)),
            pl.BlockSpec((HID, HID), lambda i: (0, 0)),
            pl.BlockSpec((HID, HID), lambda i: (0, 0)),
            pl.BlockSpec((HID, HID), lambda i: (0, 0)),
            pl.BlockSpec((1, HID), lambda i: (0, 0)),
        ],
        out_specs=pl.BlockSpec((_B, HID), lambda i: (i, 0)),
        out_shape=jax.ShapeDtypeStruct((N, HID), jnp.float32),
    )(h, u1, u2, dis, wm, wa, wb, b)


def kernel(x, edge_index, W1, b1, W2, b2):
    row = edge_index[0].astype(jnp.int32)
    col = edge_index[1].astype(jnp.int32)
    e2 = E // 2
    pes = LES - e2
    padr1 = jnp.arange(pes, dtype=jnp.int32) % N
    padc1 = N + jnp.arange(pes, dtype=jnp.int32) % 128
    rowg1 = jnp.concatenate([row[:e2], padr1, row[e2:], padr1])
    colg1 = jnp.concatenate([col[:e2], padc1, col[e2:], padc1])
    colg1 = colg1.reshape(-1, CHUNK)
    pcs = LCS - E
    padr2 = jnp.arange(pcs, dtype=jnp.int32) % N
    padc2 = N + jnp.arange(pcs, dtype=jnp.int32) % 128
    rowg2 = jnp.concatenate([row, padr2, row + N, padr2 + N])
    colg2 = jnp.concatenate([col, padc2, col, padc2]).reshape(-1, CHUNK)

    ones128 = jnp.ones((CHUNK, DC), jnp.float32)
    zeros128 = jnp.zeros((CHUNK, DC), jnp.float32)

    rowsc = jnp.concatenate([row[:e2], padc1, row[e2:], padc1])
    rowsc = rowsc.reshape(-1, CHUNK)
    degp = _deg_kernel(rowsc, ones128, zeros128)
    degp = degp.reshape(NC, NPAD, DC)

    dis, s0 = _prep(degp, x)

    u1 = _prop_es(s0, rowg1, colg1, zeros128)
    u1 = u1.reshape(2, NPAD, DC)[:, :N, :]
    s1 = _mid1(u1, dis)
    u2 = _prop_es(s1, rowg1, colg1, zeros128)
    u2 = u2.reshape(2, NPAD, DC)[:, :N, :]

    w1m = W1[0] - W1[2]
    h, s0p = _layer1(x, u1, u2, dis, w1m, W1[1], W1[2], b1.reshape(1, HID))

    u1p = _prop_cs(s0p.reshape(NC * N, DC), rowg2, colg2, zeros128)
    u1p = u1p.reshape(2, NPAD, DC)[:, :N, :]
    s1p = _mid2(u1p, dis)
    u2p = _prop_cs(s1p.reshape(NC * N, DC), rowg2, colg2, zeros128)
    u2p = u2p.reshape(2, NPAD, DC)[:, :N, :]

    w2m = W2[0] - W2[2]
    out = _layer2(h, u1p, u2p, dis, w2m, W2[1], W2[2], b2.reshape(1, HID))
    return out


# speedup vs baseline: 13.3652x; 1.0653x over previous
import functools

import jax
import jax.numpy as jnp
from jax import lax
from jax.experimental import pallas as pl
from jax.experimental.pallas import tpu as pltpu
from jax.experimental.pallas import tpu_sc as plsc

N = 10000
E = 320000
D_IN = 128
HID = 256
DC = 128
NC, NS = 2, 16
CHUNK = 128
NPAD = 10240
RPT = NPAD // NS


def _mesh():
    return plsc.VectorSubcoreMesh(core_axis_name="c", subcore_axis_name="s")


_EPT_D = 80 * CHUNK


_HR = NPAD // CHUNK


@functools.partial(
    pl.kernel,
    out_type=jax.ShapeDtypeStruct((NC * _HR, CHUNK), jnp.float32),
    mesh=_mesh(),
    compiler_params=pltpu.CompilerParams(needs_layout_passes=False),
    scratch_types=[
        pltpu.VMEM((_EPT_D,), jnp.int32),
        pltpu.VMEM((_HR, CHUNK), jnp.float32),
        pltpu.VMEM((_HR,), jnp.int32),
        pltpu.VMEM_SHARED((_HR, CHUNK), jnp.float32),
    ],
)
def _deg_kernel(rowsc_hbm, out_hbm, idxf, hist, rid, acc1):
    c = lax.axis_index("c")
    s = lax.axis_index("s")
    zeros = jnp.zeros((16,), jnp.float32)
    ones = jnp.ones((16,), jnp.float32)

    def zb(i, _):
        for v in range(CHUNK // 16):
            hist[i, pl.ds(v * 16, 16)] = zeros
        return 0

    lax.fori_loop(0, _HR, zb, 0)
    for v in range(_HR // 16):
        rid[pl.ds(v * 16, 16)] = lax.iota(jnp.int32, 16) + v * 16

    @pl.when(s == 0)
    def _():
        pltpu.sync_copy(hist, acc1)

    tb = (c * NS + s) * _EPT_D
    pltpu.sync_copy(rowsc_hbm.at[pl.ds(tb, _EPT_D)], idxf)
    plsc.subcore_barrier()

    def body(i, _):
        iv = idxf[pl.ds(i * 16, 16)]
        plsc.addupdate_scatter(hist, [iv >> 7, iv & 127], ones)
        return 0

    lax.fori_loop(0, _EPT_D // 16, body, 0)
    pltpu.sync_copy(hist, acc1.at[rid], add=True)
    plsc.subcore_barrier()

    @pl.when(s == 0)
    def _():
        pltpu.sync_copy(acc1, out_hbm.at[pl.ds(c * _HR, _HR)])


NB = 2
NG = 16


def _make_prop(cpt):
    @functools.partial(
        pl.kernel,
        out_type=jax.ShapeDtypeStruct((NC * NPAD, DC), jnp.float32),
        mesh=_mesh(),
        scratch_types=[
            pltpu.VMEM((NG * CHUNK,), jnp.int32),
            pltpu.VMEM((NG, CHUNK), jnp.int32),
            pltpu.VMEM((CHUNK, DC), jnp.float32),
            pltpu.VMEM((CHUNK, DC), jnp.float32),
            pltpu.VMEM_SHARED((NPAD, DC), jnp.float32),
            pltpu.SemaphoreType.DMA,
            pltpu.SemaphoreType.DMA,
            pltpu.SemaphoreType.DMA,
            pltpu.SemaphoreType.DMA,
        ],
    )
    def prop(table_hbm, rowg_hbm, colg_hbm, zeros_hbm, out_hbm,
             idxr_v, idxc_v, st0, st1, acc, g0, g1, s0, s1):
        stags = (st0, st1)
        gsems = (g0, g1)
        ssems = (s0, s1)
        c = lax.axis_index("c")
        s = lax.axis_index("s")
        pltpu.sync_copy(zeros_hbm, st0)
        r0 = s * RPT
        for t in range(5):
            pltpu.sync_copy(st0, acc.at[pl.ds(r0 + t * CHUNK, CHUNK)])
        tb = (c * NS + s) * cpt
        plsc.subcore_barrier()

        def gather(j, b):
            pltpu.async_copy(
                table_hbm.at[idxr_v.at[pl.ds(j * CHUNK, CHUNK)]],
                stags[b], gsems[b])

        def scatter(j, b):
            pltpu.async_copy(stags[b], acc.at[idxc_v.at[j]], ssems[b],
                             add=True)

        def drain(sem, b):
            pltpu.make_async_copy(zeros_hbm, stags[b], sem).wait()

        def group(g, _):
            gb = tb + g * NG
            pltpu.sync_copy(rowg_hbm.at[pl.ds(gb * CHUNK, NG * CHUNK)],
                            idxr_v)
            pltpu.sync_copy(colg_hbm.at[pl.ds(gb, NG)], idxc_v)
            for b in range(NB):
                gather(b, b)
            for j in range(NG):
                b = j % NB
                drain(gsems[b], b)
                scatter(j, b)
                if j + NB < NG:
                    drain(ssems[b], b)
                    gather(j + NB, b)
            for b in range(NB):
                drain(ssems[b], b)
            return 0

        lax.fori_loop(0, cpt // NG, group, 0)
        plsc.subcore_barrier()

        ob = c * NPAD + r0
        for t in range(5):
            pltpu.sync_copy(acc.at[pl.ds(r0 + t * CHUNK, CHUNK)], st0)
            pltpu.sync_copy(st0, out_hbm.at[pl.ds(ob + t * CHUNK, CHUNK)])

    return prop


CPT_ES = 80
CPT_CS = 160
LES = CPT_ES * CHUNK * NS
LCS = CPT_CS * CHUNK * NS
_prop_es = _make_prop(CPT_ES)
_prop_cs = _make_prop(CPT_CS)


_B = 1000
_HP = jax.lax.Precision.HIGHEST


def _prep_body(degp_ref, x_ref, dis_ref, s0_ref):
    deg = degp_ref[0] + degp_ref[1]
    dis = jnp.where(deg > 0, lax.rsqrt(deg), 0.0)
    dis_ref[...] = dis
    s0_ref[...] = dis * x_ref[...]


def _prep(degp, x):
    return pl.pallas_call(
        _prep_body,
        grid=(N // _B,),
        in_specs=[
            pl.BlockSpec((2, _B, 1), lambda i: (0, i, 0)),
            pl.BlockSpec((_B, D_IN), lambda i: (i, 0)),
        ],
        out_specs=[
            pl.BlockSpec((_B, 1), lambda i: (i, 0)),
            pl.BlockSpec((_B, D_IN), lambda i: (i, 0)),
        ],
        out_shape=[
            jax.ShapeDtypeStruct((N, 1), jnp.float32),
            jax.ShapeDtypeStruct((N, D_IN), jnp.float32),
        ],
    )(degp, x)


def _mid1_body(u_ref, dis_ref, s_ref):
    dis = dis_ref[...]
    s_ref[...] = (-(dis * dis)) * (u_ref[0] + u_ref[1])


def _mid1(u, dis):
    return pl.pallas_call(
        _mid1_body,
        grid=(N // _B,),
        in_specs=[
            pl.BlockSpec((2, _B, DC), lambda i: (0, i, 0)),
            pl.BlockSpec((_B, 1), lambda i: (i, 0)),
        ],
        out_specs=pl.BlockSpec((_B, DC), lambda i: (i, 0)),
        out_shape=jax.ShapeDtypeStruct((N, DC), jnp.float32),
    )(u, dis)


def _mid2_body(u_ref, dis_ref, s_ref):
    dis = dis_ref[...]
    s_ref[0] = (-(dis * dis)) * u_ref[0]


def _mid2(u, dis):
    return pl.pallas_call(
        _mid2_body,
        grid=(2, N // _B),
        in_specs=[
            pl.BlockSpec((1, _B, DC), lambda c, i: (c, i, 0)),
            pl.BlockSpec((_B, 1), lambda c, i: (i, 0)),
        ],
        out_specs=pl.BlockSpec((1, _B, DC), lambda c, i: (c, i, 0)),
        out_shape=jax.ShapeDtypeStruct((2, N, DC), jnp.float32),
    )(u, dis)


def _layer1_body(x_ref, u1_ref, u2_ref, dis_ref, wm_ref, wa_ref, wb_ref,
                 b_ref, out_ref, sp_ref):
    dis = dis_ref[...]
    u1 = u1_ref[0] + u1_ref[1]
    u2 = u2_ref[0] + u2_ref[1]
    acc = jnp.dot(x_ref[...], wm_ref[...], precision=_HP,
                  preferred_element_type=jnp.float32)
    acc = acc + jnp.dot(-dis * u1, wa_ref[...], precision=_HP,
                        preferred_element_type=jnp.float32)
    acc = acc + jnp.dot(-2.0 * dis * u2, wb_ref[...], precision=_HP,
                        preferred_element_type=jnp.float32)
    h = jnp.maximum(acc + b_ref[...], 0.0)
    out_ref[...] = h
    sp_ref[0] = dis * h[:, :DC]
    sp_ref[1] = dis * h[:, DC:]


def _layer1(x, u1, u2, dis, wm, wa, wb, b):
    return pl.pallas_call(
        _layer1_body,
        grid=(N // _B,),
        in_specs=[
            pl.BlockSpec((_B, D_IN), lambda i: (i, 0)),
            pl.BlockSpec((2, _B, DC), lambda i: (0, i, 0)),
            pl.BlockSpec((2, _B, DC), lambda i: (0, i, 0)),
            pl.BlockSpec((_B, 1), lambda i: (i, 0)),
            pl.BlockSpec((D_IN, HID), lambda i: (0, 0)),
            pl.BlockSpec((D_IN, HID), lambda i: (0, 0)),
            pl.BlockSpec((D_IN, HID), lambda i: (0, 0)),
            pl.BlockSpec((1, HID), lambda i: (0, 0)),
        ],
        out_specs=[
            pl.BlockSpec((_B, HID), lambda i: (i, 0)),
            pl.BlockSpec((2, _B, DC), lambda i: (0, i, 0)),
        ],
        out_shape=[
            jax.ShapeDtypeStruct((N, HID), jnp.float32),
            jax.ShapeDtypeStruct((2, N, DC), jnp.float32),
        ],
    )(x, u1, u2, dis, wm, wa, wb, b)


def _layer2_body(h_ref, u1_ref, u2_ref, dis_ref, wm_ref, wa_ref, wb_ref,
                 b_ref, out_ref):
    dis = dis_ref[...]
    u1c = jnp.concatenate([u1_ref[0], u1_ref[1]], axis=1)
    u2c = jnp.concatenate([u2_ref[0], u2_ref[1]], axis=1)
    acc = jnp.dot(h_ref[...], wm_ref[...], precision=_HP,
                  preferred_element_type=jnp.float32)
    acc = acc + jnp.dot(-dis * u1c, wa_ref[...], precision=_HP,
                        preferred_element_type=jnp.float32)
    acc = acc + jnp.dot(-2.0 * dis * u2c, wb_ref[...], precision=_HP,
                        preferred_element_type=jnp.float32)
    out_ref[...] = jnp.maximum(acc + b_ref[...], 0.0)


def _layer2(h, u1, u2, dis, wm, wa, wb, b):
    return pl.pallas_call(
        _layer2_body,
        grid=(N // _B,),
        in_specs=[
            pl.BlockSpec((_B, HID), lambda i: (i, 0)),
            pl.BlockSpec((2, _B, DC), lambda i: (0, i, 0)),
            pl.BlockSpec((2, _B, DC), lambda i: (0, i, 0)),
            pl.BlockSpec((_B, 1), lambda i: (i, 0)),
            pl.BlockSpec((HID, HID), lambda i: (0, 0)),
            pl.BlockSpec((HID, HID), lambda i: (0, 0)),
            pl.BlockSpec((HID, HID), lambda i: (0, 0)),
            pl.BlockSpec((1, HID), lambda i: (0, 0)),
        ],
        out_specs=pl.BlockSpec((_B, HID), lambda i: (i, 0)),
        out_shape=jax.ShapeDtypeStruct((N, HID), jnp.float32),
    )(h, u1, u2, dis, wm, wa, wb, b)


def kernel(x, edge_index, W1, b1, W2, b2):
    row = edge_index[0].astype(jnp.int32)
    col = edge_index[1].astype(jnp.int32)
    e2 = E // 2
    pes = LES - e2
    padr1 = jnp.arange(pes, dtype=jnp.int32) % N
    padc1 = N + jnp.arange(pes, dtype=jnp.int32) % 128
    rowg1 = jnp.concatenate([row[:e2], padr1, row[e2:], padr1])
    colg1 = jnp.concatenate([col[:e2], padc1, col[e2:], padc1])
    colg1 = colg1.reshape(-1, CHUNK)
    pcs = LCS - E
    padr2 = jnp.arange(pcs, dtype=jnp.int32) % N
    padc2 = N + jnp.arange(pcs, dtype=jnp.int32) % 128
    rowg2 = jnp.concatenate([row, padr2, row + N, padr2 + N])
    colg2 = jnp.concatenate([col, padc2, col, padc2]).reshape(-1, CHUNK)

    ones128 = jnp.ones((CHUNK, DC), jnp.float32)
    zeros128 = jnp.zeros((CHUNK, DC), jnp.float32)

    rowsc = jnp.concatenate([row[:e2], padc1, row[e2:], padc1])
    degp = _deg_kernel(rowsc)
    degp = degp.reshape(NC, NPAD, 1)

    dis, s0 = _prep(degp, x)

    u1 = _prop_es(s0, rowg1, colg1, zeros128)
    u1 = u1.reshape(2, NPAD, DC)[:, :N, :]
    s1 = _mid1(u1, dis)
    u2 = _prop_es(s1, rowg1, colg1, zeros128)
    u2 = u2.reshape(2, NPAD, DC)[:, :N, :]

    w1m = W1[0] - W1[2]
    h, s0p = _layer1(x, u1, u2, dis, w1m, W1[1], W1[2], b1.reshape(1, HID))

    u1p = _prop_cs(s0p.reshape(NC * N, DC), rowg2, colg2, zeros128)
    u1p = u1p.reshape(2, NPAD, DC)[:, :N, :]
    s1p = _mid2(u1p, dis)
    u2p = _prop_cs(s1p.reshape(NC * N, DC), rowg2, colg2, zeros128)
    u2p = u2p.reshape(2, NPAD, DC)[:, :N, :]

    w2m = W2[0] - W2[2]
    out = _layer2(h, u1p, u2p, dis, w2m, W2[1], W2[2], b2.reshape(1, HID))
    return out


# speedup vs baseline: 14.0192x; 1.0489x over previous
import functools

import jax
import jax.numpy as jnp
from jax import lax
from jax.experimental import pallas as pl
from jax.experimental.pallas import tpu as pltpu
from jax.experimental.pallas import tpu_sc as plsc

N = 10000
E = 320000
D_IN = 128
HID = 256
DC = 128
NC, NS = 2, 16
CHUNK = 128
NPAD = 10240
RPT = NPAD // NS


def _mesh():
    return plsc.VectorSubcoreMesh(core_axis_name="c", subcore_axis_name="s")


_EPT_D = 80 * CHUNK


_HR = NPAD // CHUNK


@functools.partial(
    pl.kernel,
    out_type=jax.ShapeDtypeStruct((NC * _HR, CHUNK), jnp.float32),
    mesh=_mesh(),
    compiler_params=pltpu.CompilerParams(needs_layout_passes=False),
    scratch_types=[
        pltpu.VMEM((_EPT_D,), jnp.int32),
        pltpu.VMEM((_HR, CHUNK), jnp.float32),
        pltpu.VMEM((_HR,), jnp.int32),
        pltpu.VMEM_SHARED((_HR, CHUNK), jnp.float32),
    ],
)
def _deg_kernel(rowsc_hbm, out_hbm, idxf, hist, rid, acc1):
    c = lax.axis_index("c")
    s = lax.axis_index("s")
    zeros = jnp.zeros((16,), jnp.float32)
    ones = jnp.ones((16,), jnp.float32)

    def zb(i, _):
        for v in range(CHUNK // 16):
            hist[i, pl.ds(v * 16, 16)] = zeros
        return 0

    lax.fori_loop(0, _HR, zb, 0)
    for v in range(_HR // 16):
        rid[pl.ds(v * 16, 16)] = lax.iota(jnp.int32, 16) + v * 16

    @pl.when(s == 0)
    def _():
        pltpu.sync_copy(hist, acc1)

    tb = (c * NS + s) * _EPT_D
    pltpu.sync_copy(rowsc_hbm.at[pl.ds(tb, _EPT_D)], idxf)
    plsc.subcore_barrier()

    def body(i, _):
        iv = idxf[pl.ds(i * 16, 16)]
        plsc.addupdate_scatter(hist, [iv >> 7, iv & 127], ones)
        return 0

    lax.fori_loop(0, _EPT_D // 16, body, 0)
    pltpu.sync_copy(hist, acc1.at[rid], add=True)
    plsc.subcore_barrier()

    @pl.when(s == 0)
    def _():
        pltpu.sync_copy(acc1, out_hbm.at[pl.ds(c * _HR, _HR)])


NB = 2


def _make_prop(cpt, ng):
    @functools.partial(
        pl.kernel,
        out_type=jax.ShapeDtypeStruct((NC * NPAD, DC), jnp.float32),
        mesh=_mesh(),
        scratch_types=[
            pltpu.VMEM((ng * CHUNK,), jnp.int32),
            pltpu.VMEM((ng, CHUNK), jnp.int32),
            pltpu.VMEM((CHUNK, DC), jnp.float32),
            pltpu.VMEM((CHUNK, DC), jnp.float32),
            pltpu.VMEM_SHARED((NPAD, DC), jnp.float32),
            pltpu.SemaphoreType.DMA,
            pltpu.SemaphoreType.DMA,
            pltpu.SemaphoreType.DMA,
            pltpu.SemaphoreType.DMA,
        ],
    )
    def prop(table_hbm, rowg_hbm, colg_hbm, zeros_hbm, out_hbm,
             idxr_v, idxc_v, st0, st1, acc, g0, g1, s0, s1):
        stags = (st0, st1)
        gsems = (g0, g1)
        ssems = (s0, s1)
        c = lax.axis_index("c")
        s = lax.axis_index("s")
        pltpu.sync_copy(zeros_hbm, st0)
        r0 = s * RPT
        for t in range(5):
            pltpu.sync_copy(st0, acc.at[pl.ds(r0 + t * CHUNK, CHUNK)])
        tb = (c * NS + s) * cpt
        plsc.subcore_barrier()

        def gather(j, b):
            pltpu.async_copy(
                table_hbm.at[idxr_v.at[pl.ds(j * CHUNK, CHUNK)]],
                stags[b], gsems[b])

        def scatter(j, b):
            pltpu.async_copy(stags[b], acc.at[idxc_v.at[j]], ssems[b],
                             add=True)

        def drain(sem, b):
            pltpu.make_async_copy(zeros_hbm, stags[b], sem).wait()

        def group(g, _):
            gb = tb + g * ng
            pltpu.sync_copy(rowg_hbm.at[pl.ds(gb * CHUNK, ng * CHUNK)],
                            idxr_v)
            pltpu.sync_copy(colg_hbm.at[pl.ds(gb, ng)], idxc_v)
            for b in range(NB):
                gather(b, b)
            for j in range(ng):
                b = j % NB
                drain(gsems[b], b)
                scatter(j, b)
                if j + NB < ng:
                    drain(ssems[b], b)
                    gather(j + NB, b)
            for b in range(NB):
                drain(ssems[b], b)
            return 0

        lax.fori_loop(0, cpt // ng, group, 0)
        plsc.subcore_barrier()

        ob = c * NPAD + r0
        for t in range(5):
            pltpu.sync_copy(acc.at[pl.ds(r0 + t * CHUNK, CHUNK)], st0)
            pltpu.sync_copy(st0, out_hbm.at[pl.ds(ob + t * CHUNK, CHUNK)])

    return prop


CPT_ES = 80
CPT_CS = 160
LES = CPT_ES * CHUNK * NS
LCS = CPT_CS * CHUNK * NS
_prop_es = _make_prop(CPT_ES, 40)
_prop_cs = _make_prop(CPT_CS, 32)


_B = 1000
_HP = jax.lax.Precision.HIGHEST


def _prep_body(degp_ref, x_ref, dis_ref, s0_ref):
    deg = degp_ref[0] + degp_ref[1]
    dis = jnp.where(deg > 0, lax.rsqrt(deg), 0.0)
    dis_ref[...] = dis
    s0_ref[...] = dis * x_ref[...]


def _prep(degp, x):
    return pl.pallas_call(
        _prep_body,
        grid=(N // _B,),
        in_specs=[
            pl.BlockSpec((2, _B, 1), lambda i: (0, i, 0)),
            pl.BlockSpec((_B, D_IN), lambda i: (i, 0)),
        ],
        out_specs=[
            pl.BlockSpec((_B, 1), lambda i: (i, 0)),
            pl.BlockSpec((_B, D_IN), lambda i: (i, 0)),
        ],
        out_shape=[
            jax.ShapeDtypeStruct((N, 1), jnp.float32),
            jax.ShapeDtypeStruct((N, D_IN), jnp.float32),
        ],
    )(degp, x)


def _mid1_body(u_ref, dis_ref, s_ref):
    dis = dis_ref[...]
    s_ref[...] = (-(dis * dis)) * (u_ref[0] + u_ref[1])


def _mid1(u, dis):
    return pl.pallas_call(
        _mid1_body,
        grid=(N // _B,),
        in_specs=[
            pl.BlockSpec((2, _B, DC), lambda i: (0, i, 0)),
            pl.BlockSpec((_B, 1), lambda i: (i, 0)),
        ],
        out_specs=pl.BlockSpec((_B, DC), lambda i: (i, 0)),
        out_shape=jax.ShapeDtypeStruct((N, DC), jnp.float32),
    )(u, dis)


def _mid2_body(u_ref, dis_ref, s_ref):
    dis = dis_ref[...]
    s_ref[0] = (-(dis * dis)) * u_ref[0]


def _mid2(u, dis):
    return pl.pallas_call(
        _mid2_body,
        grid=(2, N // _B),
        in_specs=[
            pl.BlockSpec((1, _B, DC), lambda c, i: (c, i, 0)),
            pl.BlockSpec((_B, 1), lambda c, i: (i, 0)),
        ],
        out_specs=pl.BlockSpec((1, _B, DC), lambda c, i: (c, i, 0)),
        out_shape=jax.ShapeDtypeStruct((2, N, DC), jnp.float32),
    )(u, dis)


def _layer1_body(x_ref, u1_ref, u2_ref, dis_ref, wm_ref, wa_ref, wb_ref,
                 b_ref, out_ref, sp_ref):
    dis = dis_ref[...]
    u1 = u1_ref[0] + u1_ref[1]
    u2 = u2_ref[0] + u2_ref[1]
    acc = jnp.dot(x_ref[...], wm_ref[...], precision=_HP,
                  preferred_element_type=jnp.float32)
    acc = acc + jnp.dot(-dis * u1, wa_ref[...], precision=_HP,
                        preferred_element_type=jnp.float32)
    acc = acc + jnp.dot(-2.0 * dis * u2, wb_ref[...], precision=_HP,
                        preferred_element_type=jnp.float32)
    h = jnp.maximum(acc + b_ref[...], 0.0)
    out_ref[...] = h
    sp_ref[0] = dis * h[:, :DC]
    sp_ref[1] = dis * h[:, DC:]


def _layer1(x, u1, u2, dis, wm, wa, wb, b):
    return pl.pallas_call(
        _layer1_body,
        grid=(N // _B,),
        in_specs=[
            pl.BlockSpec((_B, D_IN), lambda i: (i, 0)),
            pl.BlockSpec((2, _B, DC), lambda i: (0, i, 0)),
            pl.BlockSpec((2, _B, DC), lambda i: (0, i, 0)),
            pl.BlockSpec((_B, 1), lambda i: (i, 0)),
            pl.BlockSpec((D_IN, HID), lambda i: (0, 0)),
            pl.BlockSpec((D_IN, HID), lambda i: (0, 0)),
            pl.BlockSpec((D_IN, HID), lambda i: (0, 0)),
            pl.BlockSpec((1, HID), lambda i: (0, 0)),
        ],
        out_specs=[
            pl.BlockSpec((_B, HID), lambda i: (i, 0)),
            pl.BlockSpec((2, _B, DC), lambda i: (0, i, 0)),
        ],
        out_shape=[
            jax.ShapeDtypeStruct((N, HID), jnp.float32),
            jax.ShapeDtypeStruct((2, N, DC), jnp.float32),
        ],
    )(x, u1, u2, dis, wm, wa, wb, b)


def _layer2_body(h_ref, u1_ref, u2_ref, dis_ref, wm_ref, wa_ref, wb_ref,
                 b_ref, out_ref):
    dis = dis_ref[...]
    u1c = jnp.concatenate([u1_ref[0], u1_ref[1]], axis=1)
    u2c = jnp.concatenate([u2_ref[0], u2_ref[1]], axis=1)
    acc = jnp.dot(h_ref[...], wm_ref[...], precision=_HP,
                  preferred_element_type=jnp.float32)
    acc = acc + jnp.dot(-dis * u1c, wa_ref[...], precision=_HP,
                        preferred_element_type=jnp.float32)
    acc = acc + jnp.dot(-2.0 * dis * u2c, wb_ref[...], precision=_HP,
                        preferred_element_type=jnp.float32)
    out_ref[...] = jnp.maximum(acc + b_ref[...], 0.0)


def _layer2(h, u1, u2, dis, wm, wa, wb, b):
    return pl.pallas_call(
        _layer2_body,
        grid=(N // _B,),
        in_specs=[
            pl.BlockSpec((_B, HID), lambda i: (i, 0)),
            pl.BlockSpec((2, _B, DC), lambda i: (0, i, 0)),
            pl.BlockSpec((2, _B, DC), lambda i: (0, i, 0)),
            pl.BlockSpec((_B, 1), lambda i: (i, 0)),
            pl.BlockSpec((HID, HID), lambda i: (0, 0)),
            pl.BlockSpec((HID, HID), lambda i: (0, 0)),
            pl.BlockSpec((HID, HID), lambda i: (0, 0)),
            pl.BlockSpec((1, HID), lambda i: (0, 0)),
        ],
        out_specs=pl.BlockSpec((_B, HID), lambda i: (i, 0)),
        out_shape=jax.ShapeDtypeStruct((N, HID), jnp.float32),
    )(h, u1, u2, dis, wm, wa, wb, b)


def kernel(x, edge_index, W1, b1, W2, b2):
    row = edge_index[0].astype(jnp.int32)
    col = edge_index[1].astype(jnp.int32)
    e2 = E // 2
    pes = LES - e2
    padr1 = jnp.arange(pes, dtype=jnp.int32) % N
    padc1 = N + jnp.arange(pes, dtype=jnp.int32) % 128
    rowg1 = jnp.concatenate([row[:e2], padr1, row[e2:], padr1])
    colg1 = jnp.concatenate([col[:e2], padc1, col[e2:], padc1])
    colg1 = colg1.reshape(-1, CHUNK)
    pcs = LCS - E
    padr2 = jnp.arange(pcs, dtype=jnp.int32) % N
    padc2 = N + jnp.arange(pcs, dtype=jnp.int32) % 128
    rowg2 = jnp.concatenate([row, padr2, row + N, padr2 + N])
    colg2 = jnp.concatenate([col, padc2, col, padc2]).reshape(-1, CHUNK)

    ones128 = jnp.ones((CHUNK, DC), jnp.float32)
    zeros128 = jnp.zeros((CHUNK, DC), jnp.float32)

    rowsc = jnp.concatenate([row[:e2], padc1, row[e2:], padc1])
    degp = _deg_kernel(rowsc)
    degp = degp.reshape(NC, NPAD, 1)

    dis, s0 = _prep(degp, x)

    u1 = _prop_es(s0, rowg1, colg1, zeros128)
    u1 = u1.reshape(2, NPAD, DC)[:, :N, :]
    s1 = _mid1(u1, dis)
    u2 = _prop_es(s1, rowg1, colg1, zeros128)
    u2 = u2.reshape(2, NPAD, DC)[:, :N, :]

    w1m = W1[0] - W1[2]
    h, s0p = _layer1(x, u1, u2, dis, w1m, W1[1], W1[2], b1.reshape(1, HID))

    u1p = _prop_cs(s0p.reshape(NC * N, DC), rowg2, colg2, zeros128)
    u1p = u1p.reshape(2, NPAD, DC)[:, :N, :]
    s1p = _mid2(u1p, dis)
    u2p = _prop_cs(s1p.reshape(NC * N, DC), rowg2, colg2, zeros128)
    u2p = u2p.reshape(2, NPAD, DC)[:, :N, :]

    w2m = W2[0] - W2[2]
    out = _layer2(h, u1p, u2p, dis, w2m, W2[1], W2[2], b2.reshape(1, HID))
    return out


# speedup vs baseline: 14.4680x; 1.0320x over previous
import functools

import jax
import jax.numpy as jnp
from jax import lax
from jax.experimental import pallas as pl
from jax.experimental.pallas import tpu as pltpu
from jax.experimental.pallas import tpu_sc as plsc

N = 10000
E = 320000
D_IN = 128
HID = 256
DC = 128
NC, NS = 2, 16
CHUNK = 128
NPAD = 10240
RPT = NPAD // NS


def _mesh():
    return plsc.VectorSubcoreMesh(core_axis_name="c", subcore_axis_name="s")


_EPT_D = 80 * CHUNK


_HR = NPAD // CHUNK


@functools.partial(
    pl.kernel,
    out_type=jax.ShapeDtypeStruct((NC * _HR, CHUNK), jnp.float32),
    mesh=_mesh(),
    compiler_params=pltpu.CompilerParams(needs_layout_passes=False),
    scratch_types=[
        pltpu.VMEM((_EPT_D,), jnp.int32),
        pltpu.VMEM((_HR, CHUNK), jnp.float32),
        pltpu.VMEM((_HR,), jnp.int32),
        pltpu.VMEM_SHARED((_HR, CHUNK), jnp.float32),
    ],
)
def _deg_kernel(rowsc_hbm, out_hbm, idxf, hist, rid, acc1):
    c = lax.axis_index("c")
    s = lax.axis_index("s")
    zeros = jnp.zeros((16,), jnp.float32)
    ones = jnp.ones((16,), jnp.float32)

    def zb(i, _):
        for v in range(CHUNK // 16):
            hist[i, pl.ds(v * 16, 16)] = zeros
        return 0

    lax.fori_loop(0, _HR, zb, 0)
    for v in range(_HR // 16):
        rid[pl.ds(v * 16, 16)] = lax.iota(jnp.int32, 16) + v * 16

    @pl.when(s == 0)
    def _():
        pltpu.sync_copy(hist, acc1)

    tb = (c * NS + s) * _EPT_D
    pltpu.sync_copy(rowsc_hbm.at[pl.ds(tb, _EPT_D)], idxf)
    plsc.subcore_barrier()

    def body(i, _):
        iv = idxf[pl.ds(i * 16, 16)]
        plsc.addupdate_scatter(hist, [iv >> 7, iv & 127], ones)
        return 0

    lax.fori_loop(0, _EPT_D // 16, body, 0)
    pltpu.sync_copy(hist, acc1.at[rid], add=True)
    plsc.subcore_barrier()

    @pl.when(s == 0)
    def _():
        pltpu.sync_copy(acc1, out_hbm.at[pl.ds(c * _HR, _HR)])


NB = 2


def _make_prop(cpt, ng):
    @functools.partial(
        pl.kernel,
        out_type=jax.ShapeDtypeStruct((NC * NPAD, DC), jnp.float32),
        mesh=_mesh(),
        scratch_types=[
            pltpu.VMEM((ng * CHUNK,), jnp.int32),
            pltpu.VMEM((ng, CHUNK), jnp.int32),
            pltpu.VMEM((CHUNK, DC), jnp.float32),
            pltpu.VMEM((CHUNK, DC), jnp.float32),
            pltpu.VMEM_SHARED((NPAD, DC), jnp.float32),
            pltpu.SemaphoreType.DMA,
            pltpu.SemaphoreType.DMA,
            pltpu.SemaphoreType.DMA,
            pltpu.SemaphoreType.DMA,
        ],
    )
    def prop(table_hbm, rowg_hbm, colg_hbm, zeros_hbm, out_hbm,
             idxr_v, idxc_v, st0, st1, acc, g0, g1, s0, s1):
        stags = (st0, st1)
        gsems = (g0, g1)
        ssems = (s0, s1)
        c = lax.axis_index("c")
        s = lax.axis_index("s")
        pltpu.sync_copy(zeros_hbm, st0)
        r0 = s * RPT
        for t in range(5):
            pltpu.sync_copy(st0, acc.at[pl.ds(r0 + t * CHUNK, CHUNK)])
        tb = (c * NS + s) * cpt
        plsc.subcore_barrier()

        def gather(j, b):
            pltpu.async_copy(
                table_hbm.at[idxr_v.at[pl.ds(j * CHUNK, CHUNK)]],
                stags[b], gsems[b])

        def scatter(j, b):
            pltpu.async_copy(stags[b], acc.at[idxc_v.at[j]], ssems[b],
                             add=True)

        def drain(sem, b):
            pltpu.make_async_copy(zeros_hbm, stags[b], sem).wait()

        def group(g, _):
            gb = tb + g * ng
            pltpu.sync_copy(rowg_hbm.at[pl.ds(gb * CHUNK, ng * CHUNK)],
                            idxr_v)
            pltpu.sync_copy(colg_hbm.at[pl.ds(gb, ng)], idxc_v)
            for b in range(NB):
                gather(b, b)
            for j in range(ng):
                b = j % NB
                drain(gsems[b], b)
                scatter(j, b)
                if j + NB < ng:
                    drain(ssems[b], b)
                    gather(j + NB, b)
            for b in range(NB):
                drain(ssems[b], b)
            return 0

        lax.fori_loop(0, cpt // ng, group, 0)
        plsc.subcore_barrier()

        ob = c * NPAD + r0
        for t in range(5):
            pltpu.sync_copy(acc.at[pl.ds(r0 + t * CHUNK, CHUNK)], st0)
            pltpu.sync_copy(st0, out_hbm.at[pl.ds(ob + t * CHUNK, CHUNK)])

    return prop


CPT_ES = 80
CPT_CS = 160
LES = CPT_ES * CHUNK * NS
LCS = CPT_CS * CHUNK * NS
_prop_es = _make_prop(CPT_ES, 40)
_prop_cs = _make_prop(CPT_CS, 32)


_B = 1000
_HP = jax.lax.Precision.DEFAULT


def _prep_body(degp_ref, x_ref, dis_ref, s0_ref):
    deg = degp_ref[0] + degp_ref[1]
    dis = jnp.where(deg > 0, lax.rsqrt(deg), 0.0)
    dis_ref[...] = dis
    s0_ref[...] = dis * x_ref[...]


def _prep(degp, x):
    return pl.pallas_call(
        _prep_body,
        grid=(N // _B,),
        in_specs=[
            pl.BlockSpec((2, _B, 1), lambda i: (0, i, 0)),
            pl.BlockSpec((_B, D_IN), lambda i: (i, 0)),
        ],
        out_specs=[
            pl.BlockSpec((_B, 1), lambda i: (i, 0)),
            pl.BlockSpec((_B, D_IN), lambda i: (i, 0)),
        ],
        out_shape=[
            jax.ShapeDtypeStruct((N, 1), jnp.float32),
            jax.ShapeDtypeStruct((N, D_IN), jnp.float32),
        ],
    )(degp, x)


def _mid1_body(u_ref, dis_ref, s_ref):
    dis = dis_ref[...]
    s_ref[...] = (-(dis * dis)) * (u_ref[0] + u_ref[1])


def _mid1(u, dis):
    return pl.pallas_call(
        _mid1_body,
        grid=(N // _B,),
        in_specs=[
            pl.BlockSpec((2, _B, DC), lambda i: (0, i, 0)),
            pl.BlockSpec((_B, 1), lambda i: (i, 0)),
        ],
        out_specs=pl.BlockSpec((_B, DC), lambda i: (i, 0)),
        out_shape=jax.ShapeDtypeStruct((N, DC), jnp.float32),
    )(u, dis)


def _mid2_body(u_ref, dis_ref, s_ref):
    dis = dis_ref[...]
    s_ref[0] = (-(dis * dis)) * u_ref[0]


def _mid2(u, dis):
    return pl.pallas_call(
        _mid2_body,
        grid=(2, N // _B),
        in_specs=[
            pl.BlockSpec((1, _B, DC), lambda c, i: (c, i, 0)),
            pl.BlockSpec((_B, 1), lambda c, i: (i, 0)),
        ],
        out_specs=pl.BlockSpec((1, _B, DC), lambda c, i: (c, i, 0)),
        out_shape=jax.ShapeDtypeStruct((2, N, DC), jnp.float32),
    )(u, dis)


def _layer1_body(x_ref, u1_ref, u2_ref, dis_ref, wm_ref, wa_ref, wb_ref,
                 b_ref, out_ref, sp_ref):
    dis = dis_ref[...]
    u1 = u1_ref[0] + u1_ref[1]
    u2 = u2_ref[0] + u2_ref[1]
    acc = jnp.dot(x_ref[...], wm_ref[...], precision=_HP,
                  preferred_element_type=jnp.float32)
    acc = acc + jnp.dot(-dis * u1, wa_ref[...], precision=_HP,
                        preferred_element_type=jnp.float32)
    acc = acc + jnp.dot(-2.0 * dis * u2, wb_ref[...], precision=_HP,
                        preferred_element_type=jnp.float32)
    h = jnp.maximum(acc + b_ref[...], 0.0)
    out_ref[...] = h
    sp_ref[0] = dis * h[:, :DC]
    sp_ref[1] = dis * h[:, DC:]


def _layer1(x, u1, u2, dis, wm, wa, wb, b):
    return pl.pallas_call(
        _layer1_body,
        grid=(N // _B,),
        in_specs=[
            pl.BlockSpec((_B, D_IN), lambda i: (i, 0)),
            pl.BlockSpec((2, _B, DC), lambda i: (0, i, 0)),
            pl.BlockSpec((2, _B, DC), lambda i: (0, i, 0)),
            pl.BlockSpec((_B, 1), lambda i: (i, 0)),
            pl.BlockSpec((D_IN, HID), lambda i: (0, 0)),
            pl.BlockSpec((D_IN, HID), lambda i: (0, 0)),
            pl.BlockSpec((D_IN, HID), lambda i: (0, 0)),
            pl.BlockSpec((1, HID), lambda i: (0, 0)),
        ],
        out_specs=[
            pl.BlockSpec((_B, HID), lambda i: (i, 0)),
            pl.BlockSpec((2, _B, DC), lambda i: (0, i, 0)),
        ],
        out_shape=[
            jax.ShapeDtypeStruct((N, HID), jnp.float32),
            jax.ShapeDtypeStruct((2, N, DC), jnp.float32),
        ],
    )(x, u1, u2, dis, wm, wa, wb, b)


def _layer2_body(h_ref, u1_ref, u2_ref, dis_ref, wm_ref, wa_ref, wb_ref,
                 b_ref, out_ref):
    dis = dis_ref[...]
    u1c = jnp.concatenate([u1_ref[0], u1_ref[1]], axis=1)
    u2c = jnp.concatenate([u2_ref[0], u2_ref[1]], axis=1)
    acc = jnp.dot(h_ref[...], wm_ref[...], precision=_HP,
                  preferred_element_type=jnp.float32)
    acc = acc + jnp.dot(-dis * u1c, wa_ref[...], precision=_HP,
                        preferred_element_type=jnp.float32)
    acc = acc + jnp.dot(-2.0 * dis * u2c, wb_ref[...], precision=_HP,
                        preferred_element_type=jnp.float32)
    out_ref[...] = jnp.maximum(acc + b_ref[...], 0.0)


def _layer2(h, u1, u2, dis, wm, wa, wb, b):
    return pl.pallas_call(
        _layer2_body,
        grid=(N // _B,),
        in_specs=[
            pl.BlockSpec((_B, HID), lambda i: (i, 0)),
            pl.BlockSpec((2, _B, DC), lambda i: (0, i, 0)),
            pl.BlockSpec((2, _B, DC), lambda i: (0, i, 0)),
            pl.BlockSpec((_B, 1), lambda i: (i, 0)),
            pl.BlockSpec((HID, HID), lambda i: (0, 0)),
            pl.BlockSpec((HID, HID), lambda i: (0, 0)),
            pl.BlockSpec((HID, HID), lambda i: (0, 0)),
            pl.BlockSpec((1, HID), lambda i: (0, 0)),
        ],
        out_specs=pl.BlockSpec((_B, HID), lambda i: (i, 0)),
        out_shape=jax.ShapeDtypeStruct((N, HID), jnp.float32),
    )(h, u1, u2, dis, wm, wa, wb, b)


def kernel(x, edge_index, W1, b1, W2, b2):
    row = edge_index[0].astype(jnp.int32)
    col = edge_index[1].astype(jnp.int32)
    e2 = E // 2
    pes = LES - e2
    padr1 = jnp.arange(pes, dtype=jnp.int32) % N
    padc1 = N + jnp.arange(pes, dtype=jnp.int32) % 128
    rowg1 = jnp.concatenate([row[:e2], padr1, row[e2:], padr1])
    colg1 = jnp.concatenate([col[:e2], padc1, col[e2:], padc1])
    colg1 = colg1.reshape(-1, CHUNK)
    pcs = LCS - E
    padr2 = jnp.arange(pcs, dtype=jnp.int32) % N
    padc2 = N + jnp.arange(pcs, dtype=jnp.int32) % 128
    rowg2 = jnp.concatenate([row, padr2, row + N, padr2 + N])
    colg2 = jnp.concatenate([col, padc2, col, padc2]).reshape(-1, CHUNK)

    ones128 = jnp.ones((CHUNK, DC), jnp.float32)
    zeros128 = jnp.zeros((CHUNK, DC), jnp.float32)

    rowsc = jnp.concatenate([row[:e2], padc1, row[e2:], padc1])
    degp = _deg_kernel(rowsc)
    degp = degp.reshape(NC, NPAD, 1)

    dis, s0 = _prep(degp, x)

    u1 = _prop_es(s0, rowg1, colg1, zeros128)
    u1 = u1.reshape(2, NPAD, DC)[:, :N, :]
    s1 = _mid1(u1, dis)
    u2 = _prop_es(s1, rowg1, colg1, zeros128)
    u2 = u2.reshape(2, NPAD, DC)[:, :N, :]

    w1m = W1[0] - W1[2]
    h, s0p = _layer1(x, u1, u2, dis, w1m, W1[1], W1[2], b1.reshape(1, HID))

    u1p = _prop_cs(s0p.reshape(NC * N, DC), rowg2, colg2, zeros128)
    u1p = u1p.reshape(2, NPAD, DC)[:, :N, :]
    s1p = _mid2(u1p, dis)
    u2p = _prop_cs(s1p.reshape(NC * N, DC), rowg2, colg2, zeros128)
    u2p = u2p.reshape(2, NPAD, DC)[:, :N, :]

    w2m = W2[0] - W2[2]
    out = _layer2(h, u1p, u2p, dis, w2m, W2[1], W2[2], b2.reshape(1, HID))
    return out
